# baseline (speedup 1.0000x reference)
import functools

import jax
import jax.numpy as jnp
from jax import lax
from jax.experimental import pallas as pl
from jax.experimental.pallas import tpu as pltpu
from jax.experimental.pallas import tpu_sc as plsc

_VOCAB = 100000
_EMBED = 32
_B = 4096
_L = 200
_N = _B * _L

_NC = 2
_NS = 16
_NW = _NC * _NS
_LBLK = 4
_ROWS = _LBLK * 128
_NBLK = _L // _LBLK
_NPAIR = _NBLK // 2


def _emb_body(table_hbm, idx_hbm, out_hbm,
              idx_b0, idx_b1, gbuf0, gbuf1, tbuf0, tbuf1, sem0, sem1):
    wid = lax.axis_index("s") * _NC + lax.axis_index("c")
    iota = lax.iota(jnp.int32, 16)
    bufs = ((idx_b0, gbuf0, tbuf0, sem0), (idx_b1, gbuf1, tbuf1, sem1))
    zeros16 = jnp.zeros((16,), jnp.int32)

    def fire(k, p):
        idx_b, gbuf, _, sem = bufs[p]
        pltpu.sync_copy(
            idx_hbm.at[pl.ds(k // 2, 1), pl.ds(wid, 1),
                       pl.ds((k % 2) * _ROWS, _ROWS)],
            idx_b)
        pltpu.async_copy(table_hbm.at[idx_b.at[0, 0]], gbuf, sem)

    def drain_transpose_store(k, p):
        idx_b, gbuf, tbuf, sem = bufs[p]
        pltpu.make_async_copy(table_hbm.at[idx_b.at[0, 0]], gbuf, sem).wait()

        @plsc.parallel_loop(0, 8, 1, unroll=2)
        def tr_body(j):
            for lq in range(_LBLK):
                rows = lq * 128 + j * 16 + iota
                for e0 in range(_EMBED):
                    evec = (e0 + iota) % _EMBED
                    v = plsc.load_gather(gbuf, [rows, evec])
                    qv = lq * 4 + evec // 8
                    cv = (evec % 8) * 128 + j * 16 + iota
                    plsc.store_scatter(tbuf, [qv, zeros16, cv], v)

        pltpu.sync_copy(
            tbuf,
            out_hbm.at[pl.ds(k * 16, 16), pl.ds(wid, 1), slice(None)])

    fire(0, 0)

    def pair(q, _):
        for p in range(2):
            k = 2 * q + p
            if p == 0:
                fire(k + 1, 1)
            else:
                @pl.when(q < _NPAIR - 1)
                def _():
                    fire(k + 1, 0)
            drain_transpose_store(k, p)
        return 0

    lax.fori_loop(0, _NPAIR, pair, 0)


_emb = functools.partial(
    pl.kernel,
    mesh=plsc.VectorSubcoreMesh(core_axis_name="c", subcore_axis_name="s"),
    out_type=jax.ShapeDtypeStruct((_L * 4, _NW, 1024), jnp.float32),
    scratch_types=[
        pltpu.VMEM((1, 1, _ROWS), jnp.int32),
        pltpu.VMEM((1, 1, _ROWS), jnp.int32),
        pltpu.VMEM((_ROWS, _EMBED), jnp.float32),
        pltpu.VMEM((_ROWS, _EMBED), jnp.float32),
        pltpu.VMEM((16, 1, 1024), jnp.float32),
        pltpu.VMEM((16, 1, 1024), jnp.float32),
        pltpu.SemaphoreType.DMA,
        pltpu.SemaphoreType.DMA,
    ],
    compiler_params=pltpu.CompilerParams(use_tc_tiling_on_sc=False,
                                         needs_layout_passes=False),
)(_emb_body)


def kernel(indices, table):
    idx3 = indices.reshape(_NW, 128, 25, 8).transpose(2, 0, 3, 1)
    idx3 = idx3.reshape(25, _NW, 1024)
    out5 = _emb(table, idx3).reshape(_L, 4, _NW, 8, 128)
    return out5.transpose(2, 4, 0, 1, 3).reshape(_B, _L, _EMBED)


# speedup vs baseline: 1.1482x; 1.1482x over previous
import functools

import jax
import jax.numpy as jnp
from jax import lax
from jax.experimental import pallas as pl
from jax.experimental.pallas import tpu as pltpu
from jax.experimental.pallas import tpu_sc as plsc

_VOCAB = 100000
_EMBED = 32
_B = 4096
_L = 200
_N = _B * _L

_NC = 2
_NS = 16
_NW = _NC * _NS
_LBLK = 4
_ROWS = _LBLK * 128
_NBLK = _L // _LBLK
_NPAIR = _NBLK // 2


def _emb_body(table_hbm, idx_hbm, out_hbm,
              idx_all, gbuf0, gbuf1, tbuf0, tbuf1, sem0, sem1):
    wid = lax.axis_index("s") * _NC + lax.axis_index("c")
    iota = lax.iota(jnp.int32, 16)
    bufs = ((gbuf0, tbuf0, sem0), (gbuf1, tbuf1, sem1))
    zeros16 = jnp.zeros((16,), jnp.int32)

    pltpu.sync_copy(idx_hbm.at[slice(None), pl.ds(wid, 1), slice(None)],
                    idx_all)

    def idx_ref(kd2, km2):
        return idx_all.at[kd2, 0, pl.ds(km2 * _ROWS, _ROWS)]

    def fire(kd2, km2, p):
        gbuf, _, sem = bufs[p]
        pltpu.async_copy(table_hbm.at[idx_ref(kd2, km2)], gbuf, sem)

    def drain_transpose_store(k, kd2, km2, p):
        gbuf, tbuf, sem = bufs[p]
        pltpu.make_async_copy(table_hbm.at[idx_ref(kd2, km2)], gbuf,
                              sem).wait()

        @plsc.parallel_loop(0, 8, 1, unroll=2)
        def tr_body(j):
            for lq in range(_LBLK):
                rows = lq * 128 + j * 16 + iota
                for e0 in range(_EMBED):
                    evec = (e0 + iota) % _EMBED
                    v = plsc.load_gather(gbuf, [rows, evec])
                    qv = lq * 4 + evec // 8
                    cv = (evec % 8) * 128 + j * 16 + iota
                    plsc.store_scatter(tbuf, [qv, zeros16, cv], v)

        pltpu.sync_copy(
            tbuf,
            out_hbm.at[pl.ds(k * 16, 16), pl.ds(wid, 1), slice(None)])

    fire(0, 0, 0)

    def pair(q, _):
        for p in range(2):
            k = 2 * q + p
            if p == 0:
                fire(q, 1, 1)
            else:
                @pl.when(q < _NPAIR - 1)
                def _():
                    fire(q + 1, 0, 0)
            drain_transpose_store(k, q, p, p)
        return 0

    lax.fori_loop(0, _NPAIR, pair, 0)


_emb = functools.partial(
    pl.kernel,
    mesh=plsc.VectorSubcoreMesh(core_axis_name="c", subcore_axis_name="s"),
    out_type=jax.ShapeDtypeStruct((_L * 4, _NW, 1024), jnp.float32),
    scratch_types=[
        pltpu.VMEM((25, 1, 1024), jnp.int32),
        pltpu.VMEM((_ROWS, _EMBED), jnp.float32),
        pltpu.VMEM((_ROWS, _EMBED), jnp.float32),
        pltpu.VMEM((16, 1, 1024), jnp.float32),
        pltpu.VMEM((16, 1, 1024), jnp.float32),
        pltpu.SemaphoreType.DMA,
        pltpu.SemaphoreType.DMA,
    ],
    compiler_params=pltpu.CompilerParams(use_tc_tiling_on_sc=False,
                                         needs_layout_passes=False),
)(_emb_body)


def kernel(indices, table):
    idx3 = indices.reshape(_NW, 128, 25, 8).transpose(2, 0, 3, 1)
    idx3 = idx3.reshape(25, _NW, 1024)
    out5 = _emb(table, idx3).reshape(_L, 4, _NW, 8, 128)
    return out5.transpose(2, 4, 0, 1, 3).reshape(_B, _L, _EMBED)
